# baseline (speedup 1.0000x reference)
import functools

import jax
import jax.numpy as jnp
from jax import lax
from jax.experimental import pallas as pl
from jax.experimental.pallas import tpu as pltpu
from jax.experimental.pallas import tpu_sc as plsc

N, E, D, H, C = 10000, 320000, 128, 256, 40
NC, NS, LAN = 2, 16, 16
NT = NC * NS
NPAD = 10240
ACC_N = 3 * NPAD
TRASH = N
F_T = 2
EPR = E // NT
CK = 2048
ECW = 5 * CK
GPC = CK // LAN
BLK = 1280
GRID = NPAD // BLK

_MESH = plsc.VectorSubcoreMesh(core_axis_name="c", subcore_axis_name="s")
_SC_PARAMS = pltpu.CompilerParams(needs_layout_passes=False)
_TC_PARAMS = pltpu.CompilerParams(fuse_transposed_lhs_in_matmul=True)
_C00 = (((0,), (0,)), ((), ()))


@functools.partial(
    pl.kernel,
    out_type=(
        jax.ShapeDtypeStruct((NT, ECW), jnp.int32),
        jax.ShapeDtypeStruct((NT, LAN), jnp.int32),
    ),
    mesh=_MESH,
    compiler_params=_SC_PARAMS,
    scratch_types=[
        pltpu.VMEM((N,), jnp.int32),
        pltpu.VMEM((N,), jnp.int32),
        pltpu.VMEM((EPR,), jnp.int32),
        pltpu.VMEM((EPR,), jnp.int32),
        pltpu.VMEM((ECW,), jnp.int32),
        pltpu.VMEM((LAN,), jnp.int32),
    ],
)
def _sc_rindex(ei_ref, y_ref, m_ref, ec_ref, cnt_ref,
               nt_v, m_v, src_v, dst_v, cp_v, nb_v):
    c = lax.axis_index("c")
    s = lax.axis_index("s")
    t = s * NC + c
    pltpu.sync_copy(ei_ref.at[0, t], src_v)
    pltpu.sync_copy(ei_ref.at[1, t], dst_v)
    pltpu.sync_copy(y_ref, nt_v)
    pltpu.sync_copy(m_ref, m_v)

    def ntb(i, _):
        sl = pl.ds(i * LAN, LAN)
        nt_v[sl] = jnp.where(m_v[sl] != 0, nt_v[sl], 2)
        return 0
    lax.fori_loop(0, N // LAN, ntb, 0)

    def fill(i, _):
        cp_v[pl.ds(i * LAN, LAN)] = jnp.full((LAN,), TRASH << 14, jnp.int32)
        return 0
    lax.fori_loop(0, ECW // LAN, fill, 0)

    def rb(j, off):
        sl = pl.ds(j * LAN, LAN)
        s16 = src_v[sl]
        nt16 = plsc.load_gather(nt_v, [s16])
        alive = nt16 < 3
        e16 = s16 | ((dst_v[sl] + nt16 * NPAD) << 14)
        plsc.store_compressed(cp_v.at[pl.ds(off, LAN)], e16, mask=alive)
        pc = plsc.all_reduce_population_count(alive)
        return off + jnp.max(pc)
    cnt = lax.fori_loop(0, EPR // LAN, rb, 0)

    pltpu.sync_copy(cp_v, ec_ref.at[t])
    nb_v[pl.ds(0, LAN)] = jnp.full((LAN,), 1, jnp.int32) * cnt
    pltpu.sync_copy(nb_v, cnt_ref.at[t])


def _make_sweep(nsw):
    @functools.partial(
        pl.kernel,
        out_type=jax.ShapeDtypeStruct((nsw, NT, F_T, ACC_N), jnp.float32),
        mesh=_MESH,
        compiler_params=_SC_PARAMS,
        scratch_types=[
            pltpu.VMEM((F_T * NPAD,), jnp.float32),
            pltpu.VMEM((F_T * ACC_N,), jnp.float32),
            pltpu.VMEM((NT, LAN), jnp.int32),
            pltpu.VMEM((2, ECW), jnp.int32),
            pltpu.SemaphoreType.DMA,
        ],
    )
    def sweep(feat_ref, ec_ref, cnt_ref, gacc_ref,
              xcol_v, acc_v, ncv, eb, sem):
        c = lax.axis_index("c")
        s = lax.axis_index("s")
        t = s * NC + c
        pltpu.sync_copy(cnt_ref, ncv)

        def cnt_of(seg):
            return jnp.max(ncv[seg, pl.ds(0, LAN)])

        def nch_of(seg):
            return jnp.maximum((cnt_of(seg) + CK - 1) // CK, 1)

        for sw in range(nsw):
            pltpu.async_copy(ec_ref.at[0, pl.ds(0, CK)], eb.at[0, pl.ds(0, CK)],
                             sem)
            for ff in range(F_T):
                pltpu.sync_copy(feat_ref.at[sw * NT + t, ff],
                                xcol_v.at[pl.ds(ff * NPAD, NPAD)])

            def zz(i, _):
                for u in range(8):
                    acc_v[pl.ds((8 * i + u) * LAN, LAN)] = jnp.zeros(
                        (LAN,), jnp.float32)
                return 0
            lax.fori_loop(0, F_T * ACC_N // LAN // 8, zz, 0)

            def issue(seg, ring, k0):
                def isb(k, _):
                    pltpu.async_copy(ec_ref.at[seg, pl.ds(k * CK, CK)],
                                     eb.at[ring, pl.ds(k * CK, CK)], sem)
                    return 0
                lax.fori_loop(k0, nch_of(seg), isb, 0)

            issue(0, 0, 1)

            def seg_loop(seg, _0):
                ring = lax.rem(seg, 2)
                nch = nch_of(seg)

                def drn(k, _):
                    pltpu.make_async_copy(
                        ec_ref.at[seg, pl.ds(k * CK, CK)],
                        eb.at[ring, pl.ds(k * CK, CK)], sem).wait()
                    return 0
                lax.fori_loop(0, nch, drn, 0)

                @pl.when(seg + 1 < NT)
                def _pref():
                    issue(seg + 1, 1 - ring, 0)

                def gp(g, _2):
                    idx = []
                    for u in range(8):
                        sl = pl.ds((8 * g + u) * LAN, LAN)
                        e16 = eb[ring, sl]
                        idx.append((e16 & 0x3FFF,
                                    lax.shift_right_logical(e16, 14)))
                    vals = []
                    for s16, _r in idx:
                        for ff in range(F_T):
                            gi = s16 if ff == 0 else s16 + ff * NPAD
                            vals.append(plsc.load_gather(xcol_v, [gi]))
                    k = 0
                    for _s, r16 in idx:
                        for ff in range(F_T):
                            si = r16 if ff == 0 else r16 + ff * ACC_N
                            plsc.addupdate_scatter(acc_v, [si], vals[k])
                            k += 1
                    return 0
                lax.fori_loop(0, (cnt_of(seg) + 8 * LAN - 1) // (8 * LAN),
                              gp, 0)
                return 0
            lax.fori_loop(0, NT, seg_loop, 0)

            for ff in range(F_T):
                pltpu.sync_copy(acc_v.at[pl.ds(ff * ACC_N, ACC_N)],
                                gacc_ref.at[sw, t, ff])
    return sweep


_sc_sweep_l1 = _make_sweep(D // (NT * F_T))
_sc_sweep_l2 = _make_sweep(H // (NT * F_T))


def _tc_xpose(x_pad):
    def body(x_r, o_r):
        o_r[...] = x_r[...].T.reshape(D // F_T, F_T, BLK)
    return pl.pallas_call(
        body,
        grid=(GRID,),
        in_specs=[pl.BlockSpec((BLK, D), lambda i: (i, 0))],
        out_specs=pl.BlockSpec((D // F_T, F_T, BLK), lambda i: (0, 0, i)),
        out_shape=jax.ShapeDtypeStruct((D // F_T, F_T, NPAD), jnp.float32),
        compiler_params=_TC_PARAMS,
    )(x_pad)


def _tc_layer1(xT, g, wa, ba, wfr, wbe, wsf, bsf_c):
    def body(x_r, g0_r, g1_r, g2_r, wa_r, ba_r, wfr_r, wbe_r, wsf_r, bsf_r,
             o_r):
        xT_b = x_r[...]
        a = jax.nn.sigmoid(
            lax.dot_general(wa_r[...], xT_b, _C00,
                            preferred_element_type=jnp.float32) + ba_r[...])
        uT = g1_r[...] + a * g2_r[...]
        vT = g0_r[...] + (1.0 - a) * g2_r[...]
        accT = (lax.dot_general(wfr_r[...], uT, _C00,
                                preferred_element_type=jnp.float32)
                + lax.dot_general(wbe_r[...], vT, _C00,
                                  preferred_element_type=jnp.float32)
                + lax.dot_general(wsf_r[...], xT_b, _C00,
                                  preferred_element_type=jnp.float32)
                + bsf_r[...])
        o_r[...] = jnp.maximum(accT, 0.0).reshape(H // F_T, F_T, BLK)

    rowD = lambda i: (0, i)
    fixed = lambda i: (0, 0)
    return pl.pallas_call(
        body,
        grid=(GRID,),
        in_specs=[pl.BlockSpec((D, BLK), rowD)] + [
            pl.BlockSpec((D, BLK), lambda i, b=b: (0, b * GRID + i))
            for b in range(3)] + [
            pl.BlockSpec((D, 1), fixed), pl.BlockSpec((1, 1), fixed),
            pl.BlockSpec((D, H), fixed), pl.BlockSpec((D, H), fixed),
            pl.BlockSpec((D, H), fixed), pl.BlockSpec((H, 1), fixed)],
        out_specs=pl.BlockSpec((H // F_T, F_T, BLK), lambda i: (0, 0, i)),
        out_shape=jax.ShapeDtypeStruct((H // F_T, F_T, NPAD), jnp.float32),
        compiler_params=_TC_PARAMS,
    )(xT, g, g, g, wa, ba, wfr, wbe, wsf, bsf_c)


def _tc_layer2(hT, g, wa, ba, wfr, wbe, wsf, bsf_c, wc_p, bc_p):
    def body(h_r, g0_r, g1_r, g2_r, wa_r, ba_r, wfr_r, wbe_r, wsf_r, bsf_r,
             wc_r, bc_r, o_r):
        hT_b = h_r[...]
        a = jax.nn.sigmoid(
            lax.dot_general(wa_r[...], hT_b, _C00,
                            preferred_element_type=jnp.float32) + ba_r[...])
        uT = g1_r[...] + a * g2_r[...]
        vT = g0_r[...] + (1.0 - a) * g2_r[...]
        accT = (lax.dot_general(wfr_r[...], uT, _C00,
                                preferred_element_type=jnp.float32)
                + lax.dot_general(wbe_r[...], vT, _C00,
                                  preferred_element_type=jnp.float32)
                + lax.dot_general(wsf_r[...], hT_b, _C00,
                                  preferred_element_type=jnp.float32)
                + bsf_r[...])
        accT = jnp.maximum(accT, 0.0)
        o_r[...] = (lax.dot_general(accT, wc_r[...], _C00,
                                    preferred_element_type=jnp.float32)
                    + bc_r[...])

    rowH = lambda i: (0, i)
    fixed = lambda i: (0, 0)
    return pl.pallas_call(
        body,
        grid=(GRID,),
        in_specs=[pl.BlockSpec((H, BLK), rowH)] + [
            pl.BlockSpec((H, BLK), lambda i, b=b: (0, b * GRID + i))
            for b in range(3)] + [
            pl.BlockSpec((H, 1), fixed), pl.BlockSpec((1, 1), fixed),
            pl.BlockSpec((H, H), fixed), pl.BlockSpec((H, H), fixed),
            pl.BlockSpec((H, H), fixed), pl.BlockSpec((H, 1), fixed),
            pl.BlockSpec((H, 128), fixed), pl.BlockSpec((1, 128), fixed)],
        out_specs=pl.BlockSpec((BLK, 128), lambda i: (i, 0)),
        out_shape=jax.ShapeDtypeStruct((NPAD, 128), jnp.float32),
        compiler_params=_TC_PARAMS,
    )(hT, g, g, g, wa, ba, wfr, wbe, wsf, bsf_c, wc_p, bc_p)


def kernel(x, edge_index, y, pmp_mask,
           W_fr1, W_be1, Wa1, ba1, Wself1, bself1,
           W_fr2, W_be2, Wa2, ba2, Wself2, bself2,
           Wc, bc):
    mask_i32 = pmp_mask.astype(jnp.int32)
    ei2 = edge_index.reshape(2, NT, EPR)

    ec, nch = _sc_rindex(ei2, y, mask_i32)

    x_pad = jnp.pad(x, ((0, NPAD - N), (0, 0)))
    xT3 = _tc_xpose(x_pad)

    gacc1 = _sc_sweep_l1(xT3, ec, nch)
    g1_2d = gacc1.reshape(D, ACC_N)

    h_T3 = _tc_layer1(xT3.reshape(D, NPAD), g1_2d, Wa1, ba1.reshape(1, 1),
                      W_fr1, W_be1, Wself1, bself1.reshape(H, 1))

    gacc2 = _sc_sweep_l2(h_T3, ec, nch)
    g2_2d = gacc2.reshape(H, ACC_N)

    wcp = jnp.pad(Wc, ((0, 0), (0, 128 - C)))
    bcp = jnp.pad(bc.reshape(1, C), ((0, 0), (0, 128 - C)))
    out_p = _tc_layer2(h_T3.reshape(H, NPAD), g2_2d, Wa2, ba2.reshape(1, 1),
                       W_fr2, W_be2, Wself2, bself2.reshape(H, 1), wcp, bcp)
    return out_p[:N, :C]


# speedup vs baseline: 1.0082x; 1.0082x over previous
import functools

import jax
import jax.numpy as jnp
from jax import lax
from jax.experimental import pallas as pl
from jax.experimental.pallas import tpu as pltpu
from jax.experimental.pallas import tpu_sc as plsc

N, E, D, H, C = 10000, 320000, 128, 256, 40
NC, NS, LAN = 2, 16, 16
NT = NC * NS
NPAD = 10240
ACC_N = 3 * NPAD
TRASH = N
F_T = 2
EPR = E // NT
CK = 2048
ECW = 5 * CK
GPC = CK // LAN
BLK = 1280
GRID = NPAD // BLK

_MESH = plsc.VectorSubcoreMesh(core_axis_name="c", subcore_axis_name="s")
_SC_PARAMS = pltpu.CompilerParams(needs_layout_passes=False)
_TC_PARAMS = pltpu.CompilerParams(fuse_transposed_lhs_in_matmul=True)
_C00 = (((0,), (0,)), ((), ()))


@functools.partial(
    pl.kernel,
    out_type=(
        jax.ShapeDtypeStruct((NT, ECW), jnp.int32),
        jax.ShapeDtypeStruct((NT, LAN), jnp.int32),
    ),
    mesh=_MESH,
    compiler_params=_SC_PARAMS,
    scratch_types=[
        pltpu.VMEM((N,), jnp.int32),
        pltpu.VMEM((N,), jnp.int32),
        pltpu.VMEM((EPR,), jnp.int32),
        pltpu.VMEM((EPR,), jnp.int32),
        pltpu.VMEM((ECW,), jnp.int32),
        pltpu.VMEM((LAN,), jnp.int32),
    ],
)
def _sc_rindex(ei_ref, y_ref, m_ref, ec_ref, cnt_ref,
               nt_v, m_v, src_v, dst_v, cp_v, nb_v):
    c = lax.axis_index("c")
    s = lax.axis_index("s")
    t = s * NC + c
    pltpu.sync_copy(ei_ref.at[0, t], src_v)
    pltpu.sync_copy(ei_ref.at[1, t], dst_v)
    pltpu.sync_copy(y_ref, nt_v)
    pltpu.sync_copy(m_ref, m_v)

    def ntb(i, _):
        sl = pl.ds(i * LAN, LAN)
        nt_v[sl] = jnp.where(m_v[sl] != 0, nt_v[sl], 2)
        return 0
    lax.fori_loop(0, N // LAN, ntb, 0)

    def fill(i, _):
        cp_v[pl.ds(i * LAN, LAN)] = jnp.full((LAN,), TRASH << 14, jnp.int32)
        return 0
    lax.fori_loop(0, ECW // LAN, fill, 0)

    def rb(j, off):
        sl = pl.ds(j * LAN, LAN)
        s16 = src_v[sl]
        nt16 = plsc.load_gather(nt_v, [s16])
        alive = nt16 < 3
        e16 = s16 | ((dst_v[sl] + nt16 * NPAD) << 14)
        plsc.store_compressed(cp_v.at[pl.ds(off, LAN)], e16, mask=alive)
        pc = plsc.all_reduce_population_count(alive)
        return off + jnp.max(pc)
    cnt = lax.fori_loop(0, EPR // LAN, rb, 0)

    pltpu.sync_copy(cp_v, ec_ref.at[t])
    nb_v[pl.ds(0, LAN)] = jnp.full((LAN,), 1, jnp.int32) * cnt
    pltpu.sync_copy(nb_v, cnt_ref.at[t])


def _make_sweep(nsw):
    @functools.partial(
        pl.kernel,
        out_type=jax.ShapeDtypeStruct((nsw, NT, F_T, ACC_N), jnp.float32),
        mesh=_MESH,
        compiler_params=_SC_PARAMS,
        scratch_types=[
            pltpu.VMEM((F_T * NPAD,), jnp.float32),
            pltpu.VMEM((F_T * ACC_N,), jnp.float32),
            pltpu.VMEM((NT, LAN), jnp.int32),
            pltpu.VMEM((2, ECW), jnp.int32),
            pltpu.SemaphoreType.DMA,
        ],
    )
    def sweep(feat_ref, ec_ref, cnt_ref, gacc_ref,
              xcol_v, acc_v, ncv, eb, sem):
        c = lax.axis_index("c")
        s = lax.axis_index("s")
        t = s * NC + c
        pltpu.sync_copy(cnt_ref, ncv)

        def cnt_of(seg):
            return jnp.max(ncv[seg, pl.ds(0, LAN)])

        def nch_of(seg):
            return jnp.maximum((cnt_of(seg) + CK - 1) // CK, 1)

        for sw in range(nsw):
            for ff in range(F_T):
                pltpu.sync_copy(feat_ref.at[sw * NT + t, ff],
                                xcol_v.at[pl.ds(ff * NPAD, NPAD)])

            def zz(i, _):
                for u in range(8):
                    acc_v[pl.ds((8 * i + u) * LAN, LAN)] = jnp.zeros(
                        (LAN,), jnp.float32)
                return 0
            lax.fori_loop(0, F_T * ACC_N // LAN // 8, zz, 0)

            def issue(seg, ring):
                def isb(k, _):
                    pltpu.async_copy(ec_ref.at[seg, pl.ds(k * CK, CK)],
                                     eb.at[ring, pl.ds(k * CK, CK)], sem)
                    return 0
                lax.fori_loop(0, nch_of(seg), isb, 0)

            issue(0, 0)

            def seg_loop(seg, _0):
                ring = lax.rem(seg, 2)
                nch = nch_of(seg)

                def drn(k, _):
                    pltpu.make_async_copy(
                        ec_ref.at[seg, pl.ds(k * CK, CK)],
                        eb.at[ring, pl.ds(k * CK, CK)], sem).wait()
                    return 0
                lax.fori_loop(0, nch, drn, 0)

                @pl.when(seg + 1 < NT)
                def _pref():
                    issue(seg + 1, 1 - ring)

                def gp(g, _2):
                    idx = []
                    for u in range(8):
                        sl = pl.ds((8 * g + u) * LAN, LAN)
                        e16 = eb[ring, sl]
                        idx.append((e16 & 0x3FFF,
                                    lax.shift_right_logical(e16, 14)))
                    vals = []
                    for s16, _r in idx:
                        for ff in range(F_T):
                            gi = s16 if ff == 0 else s16 + ff * NPAD
                            vals.append(plsc.load_gather(xcol_v, [gi]))
                    k = 0
                    for _s, r16 in idx:
                        for ff in range(F_T):
                            si = r16 if ff == 0 else r16 + ff * ACC_N
                            plsc.addupdate_scatter(acc_v, [si], vals[k])
                            k += 1
                    return 0
                lax.fori_loop(0, (cnt_of(seg) + 8 * LAN - 1) // (8 * LAN),
                              gp, 0)
                return 0
            lax.fori_loop(0, NT, seg_loop, 0)

            for ff in range(F_T):
                pltpu.sync_copy(acc_v.at[pl.ds(ff * ACC_N, ACC_N)],
                                gacc_ref.at[sw, t, ff])
    return sweep


_sc_sweep_l1 = _make_sweep(D // (NT * F_T))
_sc_sweep_l2 = _make_sweep(H // (NT * F_T))


def _tc_xpose(x_pad):
    def body(x_r, o_r):
        o_r[...] = x_r[...].T.reshape(D // F_T, F_T, BLK)
    return pl.pallas_call(
        body,
        grid=(GRID,),
        in_specs=[pl.BlockSpec((BLK, D), lambda i: (i, 0))],
        out_specs=pl.BlockSpec((D // F_T, F_T, BLK), lambda i: (0, 0, i)),
        out_shape=jax.ShapeDtypeStruct((D // F_T, F_T, NPAD), jnp.float32),
        compiler_params=_TC_PARAMS,
    )(x_pad)


def _tc_layer1(xT, g, wa, ba, wfr, wbe, wsf, bsf_c):
    def body(x_r, g0_r, g1_r, g2_r, wa_r, ba_r, wfr_r, wbe_r, wsf_r, bsf_r,
             o_r):
        xT_b = x_r[...]
        a = jax.nn.sigmoid(
            lax.dot_general(wa_r[...], xT_b, _C00,
                            preferred_element_type=jnp.float32) + ba_r[...])
        uT = g1_r[...] + a * g2_r[...]
        vT = g0_r[...] + (1.0 - a) * g2_r[...]
        accT = (lax.dot_general(wfr_r[...], uT, _C00,
                                preferred_element_type=jnp.float32)
                + lax.dot_general(wbe_r[...], vT, _C00,
                                  preferred_element_type=jnp.float32)
                + lax.dot_general(wsf_r[...], xT_b, _C00,
                                  preferred_element_type=jnp.float32)
                + bsf_r[...])
        o_r[...] = jnp.maximum(accT, 0.0).reshape(H // F_T, F_T, BLK)

    rowD = lambda i: (0, i)
    fixed = lambda i: (0, 0)
    return pl.pallas_call(
        body,
        grid=(GRID,),
        in_specs=[pl.BlockSpec((D, BLK), rowD)] + [
            pl.BlockSpec((D, BLK), lambda i, b=b: (0, b * GRID + i))
            for b in range(3)] + [
            pl.BlockSpec((D, 1), fixed), pl.BlockSpec((1, 1), fixed),
            pl.BlockSpec((D, H), fixed), pl.BlockSpec((D, H), fixed),
            pl.BlockSpec((D, H), fixed), pl.BlockSpec((H, 1), fixed)],
        out_specs=pl.BlockSpec((H // F_T, F_T, BLK), lambda i: (0, 0, i)),
        out_shape=jax.ShapeDtypeStruct((H // F_T, F_T, NPAD), jnp.float32),
        compiler_params=_TC_PARAMS,
    )(xT, g, g, g, wa, ba, wfr, wbe, wsf, bsf_c)


def _tc_layer2(hT, g, wa, ba, wfr, wbe, wsf, bsf_c, wc_p, bc_p):
    def body(h_r, g0_r, g1_r, g2_r, wa_r, ba_r, wfr_r, wbe_r, wsf_r, bsf_r,
             wc_r, bc_r, o_r):
        hT_b = h_r[...]
        a = jax.nn.sigmoid(
            lax.dot_general(wa_r[...], hT_b, _C00,
                            preferred_element_type=jnp.float32) + ba_r[...])
        uT = g1_r[...] + a * g2_r[...]
        vT = g0_r[...] + (1.0 - a) * g2_r[...]
        accT = (lax.dot_general(wfr_r[...], uT, _C00,
                                preferred_element_type=jnp.float32)
                + lax.dot_general(wbe_r[...], vT, _C00,
                                  preferred_element_type=jnp.float32)
                + lax.dot_general(wsf_r[...], hT_b, _C00,
                                  preferred_element_type=jnp.float32)
                + bsf_r[...])
        accT = jnp.maximum(accT, 0.0)
        o_r[...] = (lax.dot_general(accT, wc_r[...], _C00,
                                    preferred_element_type=jnp.float32)
                    + bc_r[...])

    rowH = lambda i: (0, i)
    fixed = lambda i: (0, 0)
    return pl.pallas_call(
        body,
        grid=(GRID,),
        in_specs=[pl.BlockSpec((H, BLK), rowH)] + [
            pl.BlockSpec((H, BLK), lambda i, b=b: (0, b * GRID + i))
            for b in range(3)] + [
            pl.BlockSpec((H, 1), fixed), pl.BlockSpec((1, 1), fixed),
            pl.BlockSpec((H, H), fixed), pl.BlockSpec((H, H), fixed),
            pl.BlockSpec((H, H), fixed), pl.BlockSpec((H, 1), fixed),
            pl.BlockSpec((H, 128), fixed), pl.BlockSpec((1, 128), fixed)],
        out_specs=pl.BlockSpec((BLK, 128), lambda i: (i, 0)),
        out_shape=jax.ShapeDtypeStruct((NPAD, 128), jnp.float32),
        compiler_params=_TC_PARAMS,
    )(hT, g, g, g, wa, ba, wfr, wbe, wsf, bsf_c, wc_p, bc_p)


def kernel(x, edge_index, y, pmp_mask,
           W_fr1, W_be1, Wa1, ba1, Wself1, bself1,
           W_fr2, W_be2, Wa2, ba2, Wself2, bself2,
           Wc, bc):
    mask_i32 = pmp_mask.astype(jnp.int32)
    ei2 = edge_index.reshape(2, NT, EPR)

    ec, nch = _sc_rindex(ei2, y, mask_i32)

    x_pad = jnp.pad(x, ((0, NPAD - N), (0, 0)))
    xT3 = _tc_xpose(x_pad)

    gacc1 = _sc_sweep_l1(xT3, ec, nch)
    g1_2d = gacc1.reshape(D, ACC_N)

    h_T3 = _tc_layer1(xT3.reshape(D, NPAD), g1_2d, Wa1, ba1.reshape(1, 1),
                      W_fr1, W_be1, Wself1, bself1.reshape(H, 1))

    gacc2 = _sc_sweep_l2(h_T3, ec, nch)
    g2_2d = gacc2.reshape(H, ACC_N)

    wcp = jnp.pad(Wc, ((0, 0), (0, 128 - C)))
    bcp = jnp.pad(bc.reshape(1, C), ((0, 0), (0, 128 - C)))
    out_p = _tc_layer2(h_T3.reshape(H, NPAD), g2_2d, Wa2, ba2.reshape(1, 1),
                       W_fr2, W_be2, Wself2, bself2.reshape(H, 1), wcp, bcp)
    return out_p[:N, :C]


# speedup vs baseline: 1.0085x; 1.0003x over previous
import functools

import jax
import jax.numpy as jnp
from jax import lax
from jax.experimental import pallas as pl
from jax.experimental.pallas import tpu as pltpu
from jax.experimental.pallas import tpu_sc as plsc

N, E, D, H, C = 10000, 320000, 128, 256, 40
NC, NS, LAN = 2, 16, 16
NT = NC * NS
NPAD = 10240
ACC_N = 3 * NPAD
TRASH = N
F_T = 2
EPR = E // NT
CK = 2048
ECW = 5 * CK
BLK = 1280
GRID = NPAD // BLK

_MESH = plsc.VectorSubcoreMesh(core_axis_name="c", subcore_axis_name="s")
_SC_PARAMS = pltpu.CompilerParams(needs_layout_passes=False)
_TC_PARAMS = pltpu.CompilerParams(fuse_transposed_lhs_in_matmul=True)
_C00 = (((0,), (0,)), ((), ()))


@functools.partial(
    pl.kernel,
    out_type=(
        jax.ShapeDtypeStruct((NT, ECW), jnp.int32),
        jax.ShapeDtypeStruct((NT, LAN), jnp.int32),
    ),
    mesh=_MESH,
    compiler_params=_SC_PARAMS,
    scratch_types=[
        pltpu.VMEM((N,), jnp.int32),
        pltpu.VMEM((N,), jnp.int32),
        pltpu.VMEM((EPR,), jnp.int32),
        pltpu.VMEM((EPR,), jnp.int32),
        pltpu.VMEM((ECW,), jnp.int32),
        pltpu.VMEM((LAN,), jnp.int32),
    ],
)
def _sc_rindex(ei_ref, y_ref, m_ref, ec_ref, cnt_ref,
               nt_v, m_v, src_v, dst_v, cp_v, nb_v):
    c = lax.axis_index("c")
    s = lax.axis_index("s")
    t = s * NC + c
    pltpu.sync_copy(ei_ref.at[0, t], src_v)
    pltpu.sync_copy(ei_ref.at[1, t], dst_v)
    pltpu.sync_copy(y_ref, nt_v)
    pltpu.sync_copy(m_ref, m_v)

    def ntb(i, _):
        sl = pl.ds(i * LAN, LAN)
        nt_v[sl] = jnp.where(m_v[sl] != 0, nt_v[sl], 2)
        return 0
    lax.fori_loop(0, N // LAN, ntb, 0)

    def fill(i, _):
        cp_v[pl.ds(i * LAN, LAN)] = jnp.full((LAN,), TRASH << 14, jnp.int32)
        return 0
    lax.fori_loop(0, ECW // LAN, fill, 0)

    def rb(j, off):
        sl = pl.ds(j * LAN, LAN)
        s16 = src_v[sl]
        nt16 = plsc.load_gather(nt_v, [s16])
        alive = nt16 < 3
        e16 = s16 | ((dst_v[sl] + nt16 * NPAD) << 14)
        plsc.store_compressed(cp_v.at[pl.ds(off, LAN)], e16, mask=alive)
        pc = plsc.all_reduce_population_count(alive)
        return off + jnp.max(pc)
    cnt = lax.fori_loop(0, EPR // LAN, rb, 0)

    pltpu.sync_copy(cp_v, ec_ref.at[t])
    nb_v[pl.ds(0, LAN)] = jnp.full((LAN,), 1, jnp.int32) * cnt
    pltpu.sync_copy(nb_v, cnt_ref.at[t])


def _make_sweep(nsw):
    @functools.partial(
        pl.kernel,
        out_type=jax.ShapeDtypeStruct((nsw, NT, F_T, ACC_N), jnp.float32),
        mesh=_MESH,
        compiler_params=_SC_PARAMS,
        scratch_types=[
            pltpu.VMEM((F_T * NPAD,), jnp.float32),
            pltpu.VMEM((F_T * ACC_N,), jnp.float32),
            pltpu.VMEM((NT, LAN), jnp.int32),
            pltpu.VMEM((2, ECW), jnp.int32),
            pltpu.SemaphoreType.DMA,
        ],
    )
    def sweep(feat_ref, ec_ref, cnt_ref, gacc_ref,
              xcol_v, acc_v, ncv, eb, sem):
        c = lax.axis_index("c")
        s = lax.axis_index("s")
        t = s * NC + c
        pltpu.sync_copy(cnt_ref, ncv)

        def cnt_of(seg):
            return jnp.max(ncv[seg, pl.ds(0, LAN)])

        def nch_of(seg):
            return jnp.maximum((cnt_of(seg) + CK - 1) // CK, 1)

        for sw in range(nsw):
            for ff in range(F_T):
                pltpu.sync_copy(feat_ref.at[sw * NT + t, ff],
                                xcol_v.at[pl.ds(ff * NPAD, NPAD)])

            def zz(i, _):
                for u in range(8):
                    acc_v[pl.ds((8 * i + u) * LAN, LAN)] = jnp.zeros(
                        (LAN,), jnp.float32)
                return 0
            lax.fori_loop(0, F_T * ACC_N // LAN // 8, zz, 0)

            def issue(seg, ring):
                def isb(k, _):
                    pltpu.async_copy(ec_ref.at[seg, pl.ds(k * CK, CK)],
                                     eb.at[ring, pl.ds(k * CK, CK)], sem)
                    return 0
                lax.fori_loop(0, nch_of(seg), isb, 0)

            issue(0, 0)

            def seg_loop(seg, _0):
                ring = lax.rem(seg, 2)
                nch = nch_of(seg)

                def drn(k, _):
                    pltpu.make_async_copy(
                        ec_ref.at[seg, pl.ds(k * CK, CK)],
                        eb.at[ring, pl.ds(k * CK, CK)], sem).wait()
                    return 0
                lax.fori_loop(0, nch, drn, 0)

                @pl.when(seg + 1 < NT)
                def _pref():
                    issue(seg + 1, 1 - ring)

                def gp(g, _2):
                    idx = []
                    for u in range(8):
                        sl = pl.ds((8 * g + u) * LAN, LAN)
                        e16 = eb[ring, sl]
                        idx.append((e16 & 0x3FFF,
                                    lax.shift_right_logical(e16, 14)))
                    vals = []
                    for s16, _r in idx:
                        for ff in range(F_T):
                            gi = s16 if ff == 0 else s16 + ff * NPAD
                            vals.append(plsc.load_gather(xcol_v, [gi]))
                    k = 0
                    for _s, r16 in idx:
                        for ff in range(F_T):
                            si = r16 if ff == 0 else r16 + ff * ACC_N
                            plsc.addupdate_scatter(acc_v, [si], vals[k])
                            k += 1
                    return 0
                lax.fori_loop(0, (cnt_of(seg) + 8 * LAN - 1) // (8 * LAN),
                              gp, 0)
                return 0
            lax.fori_loop(0, NT, seg_loop, 0)

            for ff in range(F_T):
                pltpu.sync_copy(acc_v.at[pl.ds(ff * ACC_N, ACC_N)],
                                gacc_ref.at[sw, t, ff])
    return sweep


_sc_sweep_l1 = _make_sweep(D // (NT * F_T))
_sc_sweep_l2 = _make_sweep(H // (NT * F_T))


def _tc_xpose(x_pad):
    def body(x_r, o_r):
        o_r[...] = x_r[...].T.reshape(D // F_T, F_T, BLK)
    return pl.pallas_call(
        body,
        grid=(GRID,),
        in_specs=[pl.BlockSpec((BLK, D), lambda i: (i, 0))],
        out_specs=pl.BlockSpec((D // F_T, F_T, BLK), lambda i: (0, 0, i)),
        out_shape=jax.ShapeDtypeStruct((D // F_T, F_T, NPAD), jnp.float32),
        compiler_params=_TC_PARAMS,
    )(x_pad)


def _tc_layer1(xT, g, wa, ba, wfr, wbe, wsf, bsf_c):
    def body(x_r, g0_r, g1_r, g2_r, wa_r, ba_r, wfr_r, wbe_r, wsf_r, bsf_r,
             o_r):
        xT_b = x_r[...]
        a = jax.nn.sigmoid(
            lax.dot_general(wa_r[...], xT_b, _C00,
                            preferred_element_type=jnp.float32) + ba_r[...])
        uT = g1_r[...] + a * g2_r[...]
        vT = g0_r[...] + (1.0 - a) * g2_r[...]
        accT = (lax.dot_general(wfr_r[...], uT, _C00,
                                preferred_element_type=jnp.float32)
                + lax.dot_general(wbe_r[...], vT, _C00,
                                  preferred_element_type=jnp.float32)
                + lax.dot_general(wsf_r[...], xT_b, _C00,
                                  preferred_element_type=jnp.float32)
                + bsf_r[...])
        o_r[...] = jnp.maximum(accT, 0.0).reshape(H // F_T, F_T, BLK)

    rowD = lambda i: (0, i)
    fixed = lambda i: (0, 0)
    return pl.pallas_call(
        body,
        grid=(GRID,),
        in_specs=[pl.BlockSpec((D, BLK), rowD)] + [
            pl.BlockSpec((D, BLK), lambda i, b=b: (0, b * GRID + i))
            for b in range(3)] + [
            pl.BlockSpec((D, 1), fixed), pl.BlockSpec((1, 1), fixed),
            pl.BlockSpec((D, H), fixed), pl.BlockSpec((D, H), fixed),
            pl.BlockSpec((D, H), fixed), pl.BlockSpec((H, 1), fixed)],
        out_specs=pl.BlockSpec((H // F_T, F_T, BLK), lambda i: (0, 0, i)),
        out_shape=jax.ShapeDtypeStruct((H // F_T, F_T, NPAD), jnp.float32),
        compiler_params=_TC_PARAMS,
    )(xT, g, g, g, wa, ba, wfr, wbe, wsf, bsf_c)


def _tc_layer2(hT, g, wa, ba, wfr, wbe, wsf, bsf_c, wc_p, bc_p):
    def body(h_r, g0_r, g1_r, g2_r, wa_r, ba_r, wfr_r, wbe_r, wsf_r, bsf_r,
             wc_r, bc_r, o_r):
        hT_b = h_r[...]
        a = jax.nn.sigmoid(
            lax.dot_general(wa_r[...], hT_b, _C00,
                            preferred_element_type=jnp.float32) + ba_r[...])
        uT = g1_r[...] + a * g2_r[...]
        vT = g0_r[...] + (1.0 - a) * g2_r[...]
        accT = (lax.dot_general(wfr_r[...], uT, _C00,
                                preferred_element_type=jnp.float32)
                + lax.dot_general(wbe_r[...], vT, _C00,
                                  preferred_element_type=jnp.float32)
                + lax.dot_general(wsf_r[...], hT_b, _C00,
                                  preferred_element_type=jnp.float32)
                + bsf_r[...])
        accT = jnp.maximum(accT, 0.0)
        o_r[...] = (lax.dot_general(accT, wc_r[...], _C00,
                                    preferred_element_type=jnp.float32)
                    + bc_r[...])

    rowH = lambda i: (0, i)
    fixed = lambda i: (0, 0)
    return pl.pallas_call(
        body,
        grid=(GRID,),
        in_specs=[pl.BlockSpec((H, BLK), rowH)] + [
            pl.BlockSpec((H, BLK), lambda i, b=b: (0, b * GRID + i))
            for b in range(3)] + [
            pl.BlockSpec((H, 1), fixed), pl.BlockSpec((1, 1), fixed),
            pl.BlockSpec((H, H), fixed), pl.BlockSpec((H, H), fixed),
            pl.BlockSpec((H, H), fixed), pl.BlockSpec((H, 1), fixed),
            pl.BlockSpec((H, 128), fixed), pl.BlockSpec((1, 128), fixed)],
        out_specs=pl.BlockSpec((BLK, 128), lambda i: (i, 0)),
        out_shape=jax.ShapeDtypeStruct((NPAD, 128), jnp.float32),
        compiler_params=_TC_PARAMS,
    )(hT, g, g, g, wa, ba, wfr, wbe, wsf, bsf_c, wc_p, bc_p)


def kernel(x, edge_index, y, pmp_mask,
           W_fr1, W_be1, Wa1, ba1, Wself1, bself1,
           W_fr2, W_be2, Wa2, ba2, Wself2, bself2,
           Wc, bc):
    mask_i32 = pmp_mask.astype(jnp.int32)
    ei2 = edge_index.reshape(2, NT, EPR)

    ec, nch = _sc_rindex(ei2, y, mask_i32)

    x_pad = jnp.pad(x, ((0, NPAD - N), (0, 0)))
    xT3 = _tc_xpose(x_pad)

    gacc1 = _sc_sweep_l1(xT3, ec, nch)
    g1_2d = gacc1.reshape(D, ACC_N)

    h_T3 = _tc_layer1(xT3.reshape(D, NPAD), g1_2d, Wa1, ba1.reshape(1, 1),
                      W_fr1, W_be1, Wself1, bself1.reshape(H, 1))

    gacc2 = _sc_sweep_l2(h_T3, ec, nch)
    g2_2d = gacc2.reshape(H, ACC_N)

    wcp = jnp.pad(Wc, ((0, 0), (0, 128 - C)))
    bcp = jnp.pad(bc.reshape(1, C), ((0, 0), (0, 128 - C)))
    out_p = _tc_layer2(h_T3.reshape(H, NPAD), g2_2d, Wa2, ba2.reshape(1, 1),
                       W_fr2, W_be2, Wself2, bself2.reshape(H, 1), wcp, bcp)
    return out_p[:N, :C]
